# baseline (speedup 1.0000x reference)
import functools

import jax
import jax.numpy as jnp
from jax import lax
from jax.experimental import pallas as pl
from jax.experimental.pallas import tpu as pltpu
from jax.experimental.pallas import tpu_sc as plsc

NC = 2
NS = 16
NW = NC * NS
LANES = 16

BATCH = 4096
MAXLEN = 200
EMBED = 64

ROWS = BATCH * MAXLEN
RPW = ROWS // NW
CHUNK = 128
NCHUNK = RPW // CHUNK
VREGS_PER_ROW = EMBED // LANES


def _emb_body(x_hbm, tok_hbm, pos2_hbm, out_hbm, idx_v, pos2_v, rows_v, sem):
    wid = lax.axis_index("s") * NC + lax.axis_index("c")

    pltpu.sync_copy(x_hbm.at[pl.ds(wid * NCHUNK, NCHUNK)], idx_v)
    pltpu.sync_copy(pos2_hbm, pos2_v)

    def chunk_body(c, carry):
        pltpu.async_copy(tok_hbm.at[idx_v.at[c]], rows_v, sem).wait()

        p = lax.rem(c * CHUNK, MAXLEN)

        def row_body(r, rcarry):
            for q in range(VREGS_PER_ROW):
                s = pl.ds(q * LANES, LANES)
                rows_v[r, s] = rows_v[r, s] + pos2_v[p + r, s]
            return rcarry

        lax.fori_loop(0, CHUNK, row_body, 0)

        base = wid * RPW + c * CHUNK
        pltpu.sync_copy(rows_v, out_hbm.at[pl.ds(base, CHUNK)])
        return carry

    lax.fori_loop(0, NCHUNK, chunk_body, 0)


@jax.jit
def kernel(x, token_table, pos_table):
    x_flat = x.reshape(ROWS // CHUNK, CHUNK)
    pos2 = jnp.concatenate([pos_table, pos_table], axis=0)

    mesh = plsc.VectorSubcoreMesh(
        core_axis_name="c", subcore_axis_name="s", num_cores=NC, num_subcores=NS
    )
    out = pl.kernel(
        _emb_body,
        out_type=jax.ShapeDtypeStruct((ROWS, EMBED), jnp.float32),
        mesh=mesh,
        compiler_params=pltpu.CompilerParams(use_tc_tiling_on_sc=False),
        scratch_types=[
            pltpu.VMEM((NCHUNK, CHUNK), jnp.int32),
            pltpu.VMEM((2 * MAXLEN, EMBED), jnp.float32),
            pltpu.VMEM((CHUNK, EMBED), jnp.float32),
            pltpu.SemaphoreType.DMA,
        ],
    )(x_flat, token_table, pos2)
    return out.reshape(BATCH, MAXLEN, EMBED)


# speedup vs baseline: 1.1756x; 1.1756x over previous
import functools

import jax
import jax.numpy as jnp
from jax import lax
from jax.experimental import pallas as pl
from jax.experimental.pallas import tpu as pltpu
from jax.experimental.pallas import tpu_sc as plsc

NC = 2
NS = 16
NW = NC * NS
LANES = 16

BATCH = 4096
MAXLEN = 200
EMBED = 64

ROWS = BATCH * MAXLEN
RPW = ROWS // NW
CHUNK = 128
NCHUNK = RPW // CHUNK
VREGS_PER_ROW = EMBED // LANES
NBUF = 4
NGROUP = NCHUNK // NBUF


def _emb_body(x_hbm, tok_hbm, pos2_hbm, out_hbm, idx_v, pos2_v,
              ibufs, obufs, gsems, osems):
    wid = lax.axis_index("s") * NC + lax.axis_index("c")

    pltpu.sync_copy(x_hbm.at[pl.ds(wid * NCHUNK, NCHUNK)], idx_v)
    pltpu.sync_copy(pos2_hbm, pos2_v)

    def gather(c, b):
        return pltpu.async_copy(tok_hbm.at[idx_v.at[c]], ibufs.at[b], gsems.at[b])

    for b in range(NBUF):
        gather(b, b)

    def group_body(g, carry):
        for b in range(NBUF):
            c = g * NBUF + b
            pltpu.make_async_copy(
                tok_hbm.at[idx_v.at[c]], ibufs.at[b], gsems.at[b]).wait()
            @pl.when(g > 0)
            def _():
                pltpu.make_async_copy(
                    obufs.at[b], out_hbm.at[pl.ds(wid * RPW, CHUNK)],
                    osems.at[b]).wait()

            p = lax.rem(c * CHUNK, MAXLEN)

            def row_body(r, rcarry):
                for q in range(VREGS_PER_ROW):
                    s = pl.ds(q * LANES, LANES)
                    obufs[b, r, s] = ibufs[b, r, s] + pos2_v[p + r, s]
                return rcarry

            lax.fori_loop(0, CHUNK, row_body, 0, unroll=2)

            @pl.when(c + NBUF < NCHUNK)
            def _():
                gather(c + NBUF, b)

            base = wid * RPW + c * CHUNK
            pltpu.async_copy(
                obufs.at[b], out_hbm.at[pl.ds(base, CHUNK)], osems.at[b])
        return carry

    lax.fori_loop(0, NGROUP, group_body, 0)

    for b in range(NBUF):
        pltpu.make_async_copy(
            obufs.at[b], out_hbm.at[pl.ds(wid * RPW, CHUNK)], osems.at[b]).wait()


@jax.jit
def kernel(x, token_table, pos_table):
    x_flat = x.reshape(ROWS // CHUNK, CHUNK)
    pos2 = jnp.concatenate([pos_table, pos_table], axis=0)

    mesh = plsc.VectorSubcoreMesh(
        core_axis_name="c", subcore_axis_name="s", num_cores=NC, num_subcores=NS
    )
    out = pl.kernel(
        _emb_body,
        out_type=jax.ShapeDtypeStruct((ROWS, EMBED), jnp.float32),
        mesh=mesh,
        compiler_params=pltpu.CompilerParams(use_tc_tiling_on_sc=False),
        scratch_types=[
            pltpu.VMEM((NCHUNK, CHUNK), jnp.int32),
            pltpu.VMEM((2 * MAXLEN, EMBED), jnp.float32),
            pltpu.VMEM((NBUF, CHUNK, EMBED), jnp.float32),
            pltpu.VMEM((NBUF, CHUNK, EMBED), jnp.float32),
            pltpu.SemaphoreType.DMA((NBUF,)),
            pltpu.SemaphoreType.DMA((NBUF,)),
        ],
    )(x_flat, token_table, pos2)
    return out.reshape(BATCH, MAXLEN, EMBED)


# speedup vs baseline: 1.8531x; 1.5763x over previous
import jax
import jax.numpy as jnp
from jax import lax
from jax.experimental import pallas as pl
from jax.experimental.pallas import tpu as pltpu
from jax.experimental.pallas import tpu_sc as plsc

NC = 2
NS = 16
NW = NC * NS

BATCH = 4096
MAXLEN = 200
EMBED = 64
PAD = 128

ROWS = BATCH * MAXLEN
RPW = ROWS // NW
CHUNK = 128
NCHUNK = RPW // CHUNK
NBUF = 2
BLK = 8
NBLK = NCHUNK // BLK


def _emb_body(x_hbm, tok_hbm, pose_hbm, iota_hbm, out_hbm,
              idxb_v, pose_v, iota_v, ibufs, obufs_sh, isems, gsems, osems):
    sid = lax.axis_index("s")
    wid = sid * NC + lax.axis_index("c")
    base = wid * RPW
    obufs = obufs_sh.at[sid]

    pltpu.sync_copy(pose_hbm, pose_v)
    pltpu.sync_copy(iota_hbm, iota_v)

    def idx_block_descr(k):
        st = pl.multiple_of(wid * NCHUNK + k * BLK, 8)
        kr = lax.rem(k, 2)
        return pltpu.make_async_copy(
            x_hbm.at[pl.ds(st, BLK)], idxb_v.at[kr], isems.at[kr])

    def gather(c):
        b = lax.rem(c, NBUF)
        kr = lax.rem(lax.div(c, BLK), 2)
        j = lax.rem(c, BLK)
        pltpu.async_copy(tok_hbm.at[idxb_v.at[kr].at[j]], ibufs.at[b],
                         gsems.at[b])

    def gather_wait(c):
        b = lax.rem(c, NBUF)
        kr = lax.rem(lax.div(c, BLK), 2)
        j = lax.rem(c, BLK)
        pltpu.make_async_copy(tok_hbm.at[idxb_v.at[kr].at[j]], ibufs.at[b],
                              gsems.at[b]).wait()

    def store_descr(c):
        b = lax.rem(c, NBUF)
        st = pl.multiple_of(base + c * CHUNK, 8)
        return pltpu.make_async_copy(
            obufs.at[b], out_hbm.at[pl.ds(st, CHUNK)], osems.at[b])

    pltpu.sync_copy(
        x_hbm.at[pl.ds(pl.multiple_of(wid * NCHUNK, 8), BLK)], idxb_v.at[0])
    idx_block_descr(1).start()
    gather(0)

    def chunk_body(c, carry):
        b = lax.rem(c, NBUF)
        j = lax.rem(c, BLK)
        kb = lax.div(c, BLK)

        @pl.when(jnp.logical_and(j == 0, jnp.logical_and(c > 0, kb + 1 < NBLK)))
        def _():
            idx_block_descr(kb + 1).start()

        @pl.when(c + 1 < NCHUNK)
        def _():
            @pl.when(j == BLK - 1)
            def _():
                idx_block_descr(kb + 1).wait()
            gather(c + 1)

        @pl.when(c >= NBUF)
        def _():
            store_descr(c - NBUF).wait()

        p0 = pl.multiple_of(lax.rem(c * CHUNK, MAXLEN), 8)
        pltpu.sync_copy(pose_v.at[pl.ds(p0, CHUNK)], obufs.at[b])

        gather_wait(c)
        pltpu.sync_copy(ibufs.at[b], obufs.at[b].at[iota_v.at[0]], add=True)

        store_descr(c).start()
        return carry

    lax.fori_loop(0, NCHUNK, chunk_body, 0)

    for k in range(NBUF):
        store_descr(NCHUNK - NBUF + k).wait()


@jax.jit
def kernel(x, token_table, pos_table):
    x_flat = x.reshape(ROWS // CHUNK, CHUNK)
    tok128 = jnp.pad(token_table, ((0, 0), (0, PAD - EMBED)))
    pose = jnp.pad(jnp.concatenate([pos_table, pos_table], axis=0),
                   ((0, 0), (0, PAD - EMBED)))
    iota = jnp.arange(CHUNK, dtype=jnp.int32).reshape(1, CHUNK)

    mesh = plsc.VectorSubcoreMesh(
        core_axis_name="c", subcore_axis_name="s", num_cores=NC, num_subcores=NS
    )
    out = pl.kernel(
        _emb_body,
        out_type=jax.ShapeDtypeStruct((ROWS, PAD), jnp.float32),
        mesh=mesh,
        scratch_types=[
            pltpu.VMEM((2, BLK, CHUNK), jnp.int32),
            pltpu.VMEM((2 * MAXLEN, PAD), jnp.float32),
            pltpu.VMEM((1, CHUNK), jnp.int32),
            pltpu.VMEM((NBUF, CHUNK, PAD), jnp.float32),
            pltpu.VMEM_SHARED((NS, NBUF, CHUNK, PAD), jnp.float32),
            pltpu.SemaphoreType.DMA((2,)),
            pltpu.SemaphoreType.DMA((NBUF,)),
            pltpu.SemaphoreType.DMA((NBUF,)),
        ],
    )(x_flat, tok128, pose, iota)
    return out[:, :EMBED].reshape(BATCH, MAXLEN, EMBED)
